# Initial kernel scaffold; baseline (speedup 1.0000x reference)
import jax
import jax.numpy as jnp
from jax.experimental import pallas as pl


def kernel(x, dw_w, dw_b, ln_g, ln_b, off_w, off_b, mask_w, mask_b, in_w, in_b, out_w, out_b):
    raise NotImplementedError("write your pallas kernel here")


# speedup vs baseline: 1.1869x; 1.1869x over previous
import functools

import jax
import jax.numpy as jnp
from jax import lax
from jax.experimental import pallas as pl
from jax.experimental.pallas import tpu as pltpu

_N, _L, _C = 2, 4096, 1024
_K, _G = 7, 4
_GC = _C // _G
_SCALE = 2.0
_LB = 1024
_NI = _L // _LB
_SQRT_HALF = 0.7071067811865476


def _prep_body(xp, xc, xn, dww, dwb, lng, lnb, inw, inb, omw, omb,
               proj_ref, om_ref):
    i = pl.program_id(1)
    x = xc[0]
    left = jnp.where(i > 0, xp[0, _LB - 1:_LB, :], 0.0)
    right = jnp.where(i < _NI - 1, xn[0, 0:1, :], 0.0)
    xm1 = jnp.concatenate([left, x[:-1]], axis=0)
    xp1 = jnp.concatenate([x[1:], right], axis=0)
    xdw = xm1 * dww[0:1] + x * dww[1:2] + xp1 * dww[2:3] + dwb[...]
    mu = jnp.mean(xdw, axis=-1, keepdims=True)
    xz = xdw - mu
    var = jnp.mean(xz * xz, axis=-1, keepdims=True)
    xdw = xz * lax.rsqrt(var + 1e-5) * lng[...] + lnb[...]
    xdw = 0.5 * xdw * (1.0 + lax.erf(xdw * _SQRT_HALF))
    proj_ref[0] = (
        lax.dot_general(x, inw[...], (((1,), (1,)), ((), ())),
                        preferred_element_type=jnp.float32) + inb[...])
    om_ref[0] = (
        lax.dot_general(omw[...], xdw, (((1,), (1,)), ((), ())),
                        preferred_element_type=jnp.float32) + omb[...])


def _outproj_body(y, w, b, o_ref):
    o_ref[...] = (
        lax.dot_general(y[...], w[...], (((1,), (1,)), ((), ())),
                        preferred_element_type=jnp.float32) + b[...])


@functools.partial(jax.jit, static_argnames=())
def kernel(x, dw_w, dw_b, ln_g, ln_b, off_w, off_b, mask_w, mask_b,
           in_w, in_b, out_w, out_b):
    n, l, c = x.shape
    dww = jnp.transpose(dw_w[:, 0, :])
    omw = jnp.concatenate([off_w, mask_w], axis=0)
    omb = jnp.concatenate([off_b, mask_b], axis=0)[:, None]

    proj, om = pl.pallas_call(
        _prep_body,
        grid=(_N, _NI),
        in_specs=[
            pl.BlockSpec((1, _LB, _C), lambda n_, i: (n_, jnp.maximum(i - 1, 0), 0)),
            pl.BlockSpec((1, _LB, _C), lambda n_, i: (n_, i, 0)),
            pl.BlockSpec((1, _LB, _C), lambda n_, i: (n_, jnp.minimum(i + 1, _NI - 1), 0)),
            pl.BlockSpec((3, _C), lambda n_, i: (0, 0)),
            pl.BlockSpec((1, _C), lambda n_, i: (0, 0)),
            pl.BlockSpec((1, _C), lambda n_, i: (0, 0)),
            pl.BlockSpec((1, _C), lambda n_, i: (0, 0)),
            pl.BlockSpec((_C, _C), lambda n_, i: (0, 0)),
            pl.BlockSpec((1, _C), lambda n_, i: (0, 0)),
            pl.BlockSpec((_G * _K * 2, _C), lambda n_, i: (0, 0)),
            pl.BlockSpec((_G * _K * 2, 1), lambda n_, i: (0, 0)),
        ],
        out_specs=[
            pl.BlockSpec((1, _LB, _C), lambda n_, i: (n_, i, 0)),
            pl.BlockSpec((1, _G * _K * 2, _LB), lambda n_, i: (n_, 0, i)),
        ],
        out_shape=[
            jax.ShapeDtypeStruct((_N, _L, _C), jnp.float32),
            jax.ShapeDtypeStruct((_N, _G * _K * 2, _L), jnp.float32),
        ],
    )(x, x, x, dww, dw_b[None], ln_g[None], ln_b[None], in_w, in_b[None],
      omw, omb)

    offsets = jnp.transpose(om[:, :_G * _K, :].reshape(n, _G, _K, l),
                            (0, 3, 1, 2)) * _SCALE
    mask = jax.nn.softmax(
        jnp.transpose(om[:, _G * _K:, :].reshape(n, _G, _K, l), (0, 3, 1, 2)),
        axis=-1)
    ref_off = jnp.linspace(-(_K // 2), _K // 2, _K).astype(x.dtype)
    pos = jnp.arange(l, dtype=x.dtype).reshape(1, l, 1)
    x_grouped = proj.reshape(n, l, _G, _GC)
    out = jnp.zeros((n, l, _G, _GC), dtype=x.dtype)
    bidx = jnp.arange(n).reshape(n, 1, 1)
    gidx = jnp.arange(_G).reshape(1, 1, _G)
    for k in range(_K):
        abs_pos = pos + ref_off[k] + offsets[:, :, :, k]
        apc = jnp.clip(abs_pos, 0, l - 1)
        p_floor = jnp.clip(apc.astype(jnp.int32), 0, l - 1)
        p_ceil = jnp.clip(p_floor + 1, 0, l - 1)
        w_ceil = apc - p_floor.astype(x.dtype)
        w_floor = 1.0 - w_ceil
        valid = jnp.logical_not((abs_pos < 0) | (abs_pos > l - 1)).astype(x.dtype)
        w_floor = w_floor * valid
        w_ceil = w_ceil * valid
        v_floor = x_grouped[bidx, p_floor, gidx]
        v_ceil = x_grouped[bidx, p_ceil, gidx]
        sampled = v_floor * w_floor[..., None] + v_ceil * w_ceil[..., None]
        out = out + sampled * mask[:, :, :, k:k + 1]
    y = out.reshape(n * l, c)

    o = pl.pallas_call(
        _outproj_body,
        grid=(n * l // _LB,),
        in_specs=[
            pl.BlockSpec((_LB, _C), lambda i: (i, 0)),
            pl.BlockSpec((_C, _C), lambda i: (0, 0)),
            pl.BlockSpec((1, _C), lambda i: (0, 0)),
        ],
        out_specs=pl.BlockSpec((_LB, _C), lambda i: (i, 0)),
        out_shape=jax.ShapeDtypeStruct((n * l, _C), jnp.float32),
    )(y, out_w, out_b[None])
    return o.reshape(n, l, c)
